# baseline (speedup 1.0000x reference)
import functools

import jax
import jax.numpy as jnp
from jax import lax
from jax.experimental import pallas as pl
from jax.experimental.pallas import tpu as pltpu
from jax.experimental.pallas import tpu_sc as plsc

_NUM_CLASSES = 8192
_Z_DIM = 256


def _make_sc_copy():
    info = plsc.get_sparse_core_info()
    nc, ns = info.num_cores, info.num_subcores
    nw = nc * ns
    rows_per_w = _NUM_CLASSES // nw
    mesh = plsc.VectorSubcoreMesh(core_axis_name="c", subcore_axis_name="s")

    @functools.partial(
        pl.kernel,
        mesh=mesh,
        out_type=jax.ShapeDtypeStruct((_NUM_CLASSES, _Z_DIM), jnp.float32),
        scratch_types=[
            pltpu.VMEM((rows_per_w, _Z_DIM), jnp.float32),
            pltpu.SemaphoreType.DMA,
        ],
    )
    def sc_copy(a_hbm, o_hbm, buf, sem):
        wid = lax.axis_index("s") * nc + lax.axis_index("c")
        base = wid * rows_per_w
        pltpu.async_copy(a_hbm.at[pl.ds(base, rows_per_w), :], buf, sem).wait()
        pltpu.async_copy(buf, o_hbm.at[pl.ds(base, rows_per_w), :], sem).wait()

    return sc_copy


_SC_COPY = _make_sc_copy()


def kernel(_, anchor):
    return _SC_COPY(anchor)


# speedup vs baseline: 3.4194x; 3.4194x over previous
import jax
import jax.numpy as jnp
from jax.experimental import pallas as pl
from jax.experimental.pallas import tpu as pltpu

_NUM_CLASSES = 8192
_Z_DIM = 256
_BLOCK_ROWS = 2752


def _copy_body(a_ref, o_ref):
    o_ref[...] = a_ref[...]


def kernel(_, anchor):
    grid = (pl.cdiv(_NUM_CLASSES, _BLOCK_ROWS),)
    return pl.pallas_call(
        _copy_body,
        grid=grid,
        in_specs=[pl.BlockSpec((_BLOCK_ROWS, _Z_DIM), lambda i: (i, 0))],
        out_specs=pl.BlockSpec((_BLOCK_ROWS, _Z_DIM), lambda i: (i, 0)),
        out_shape=jax.ShapeDtypeStruct((_NUM_CLASSES, _Z_DIM), jnp.float32),
        compiler_params=pltpu.CompilerParams(dimension_semantics=("arbitrary",)),
    )(anchor)


# speedup vs baseline: 4.1837x; 1.2235x over previous
import jax
import jax.numpy as jnp
from jax.experimental import pallas as pl
from jax.experimental.pallas import tpu as pltpu

_NUM_CLASSES = 8192
_Z_DIM = 256
_BLOCK_ROWS = 4096


def _copy_body(a_ref, o_ref):
    o_ref[...] = a_ref[...]


def kernel(_, anchor):
    grid = (_NUM_CLASSES // _BLOCK_ROWS,)
    return pl.pallas_call(
        _copy_body,
        grid=grid,
        in_specs=[pl.BlockSpec((_BLOCK_ROWS, _Z_DIM), lambda i: (i, 0))],
        out_specs=pl.BlockSpec((_BLOCK_ROWS, _Z_DIM), lambda i: (i, 0)),
        out_shape=jax.ShapeDtypeStruct((_NUM_CLASSES, _Z_DIM), jnp.float32),
        compiler_params=pltpu.CompilerParams(dimension_semantics=("arbitrary",)),
    )(anchor)
